# baseline (speedup 1.0000x reference)
import jax
import jax.numpy as jnp
from jax.experimental import pallas as pl

B = 32
OBS = 72
SEQ = 96
N = 21
NP_ = 24
DM = 128
DFF = 256
L = 2
E = 4
EP = 8
K = 2
LPRED = 96
RT = NP_ * SEQ
RD = LPRED * NP_

_F = jnp.float32
_HI = jax.lax.Precision.HIGHEST


def _moe_dec_kernel(x_ref, tt_ref, wstart_ref, bstart_ref, gw_ref, gb_ref,
                    w1_ref, b1_ref, w2_ref, b2_ref,
                    sw_ref, sb_ref, pw_ref, pb_ref,
                    dw1_ref, db1_ref, dw2_ref, db2_ref,
                    dw3_ref, db3_ref,
                    out_ref):
    x = x_ref[0]
    tok = x * wstart_ref[...] + bstart_ref[...]

    for l in range(L):
        logits = jnp.dot(tok, gw_ref[l], preferred_element_type=_F) + gb_ref[...]
        lane = jax.lax.broadcasted_iota(jnp.int32, (RT, EP), 1)
        m1 = jnp.max(logits, axis=1, keepdims=True)
        i1 = jnp.min(jnp.where(logits == m1, lane, EP), axis=1, keepdims=True)
        is1 = lane == i1
        l2 = jnp.where(is1, -1e30, logits)
        m2 = jnp.max(l2, axis=1, keepdims=True)
        i2 = jnp.min(jnp.where(l2 == m2, lane, EP), axis=1, keepdims=True)
        is2 = lane == i2
        ex = jnp.exp(m2 - m1)
        den = 1.0 + ex
        gates = ((1.0 / den) * is1.astype(_F)
                 + (ex / den) * is2.astype(_F))

        h = jnp.maximum(jnp.dot(tok, w1_ref[l], preferred_element_type=_F)
                        + b1_ref[l], 0.0)
        y = None
        for e in range(E):
            ye = jnp.dot(h[:, e * DFF:(e + 1) * DFF], w2_ref[l, e],
                         preferred_element_type=_F) + b2_ref[l, e]
            ye = gates[:, e:e + 1] * ye
            y = ye if y is None else y + ye
        tok = tok + y

    enc = jnp.sum(tok.reshape(NP_, SEQ, DM), axis=1) * (1.0 / SEQ)

    tt = tt_ref[0]
    lane = jax.lax.broadcasted_iota(jnp.int32, (LPRED, DM), 1)
    te = jnp.where(lane == 0, tt * sw_ref[...] + sb_ref[...],
                   jnp.sin(tt * pw_ref[...] + pb_ref[...]))

    encf = jnp.concatenate(
        [jnp.broadcast_to(enc[None, :, :], (LPRED, NP_, DM)).reshape(RD, DM),
         jnp.broadcast_to(te[:, None, :], (LPRED, NP_, DM)).reshape(RD, DM)],
        axis=1)
    h1 = jnp.maximum(jnp.dot(encf, dw1_ref[...], preferred_element_type=_F)
                     + db1_ref[...], 0.0)
    h2 = jnp.maximum(jnp.dot(h1, dw2_ref[...], preferred_element_type=_F)
                     + db2_ref[...], 0.0)
    o = (jnp.dot(h2, dw3_ref[...], preferred_element_type=_F)
         + db3_ref[...])
    out_ref[0] = o


def kernel(tp_to_predict, observed_data, observed_tp, observed_mask, W_start,
           b_start, gate_W, e_W1, e_b1, e_W2, e_b2, te_scale_W, te_scale_b,
           te_per_W, te_per_b, dec_W1, dec_b1, dec_W2, dec_b2, dec_W3, dec_b3):
    x = jnp.pad(observed_data, ((0, 0), (0, SEQ - OBS), (0, 0)))
    x = jnp.pad(x.transpose(0, 2, 1), ((0, 0), (0, NP_ - N), (0, 0)))
    x = x.reshape(B, RT, 1)
    tt = tp_to_predict.reshape(B, LPRED, 1)

    gw = jnp.concatenate([gate_W, jnp.zeros((L, DM, EP - E), _F)], axis=2)
    gb = jnp.concatenate([jnp.zeros((1, E), _F),
                          jnp.full((1, EP - E), -1e30, _F)], axis=1)
    w1 = e_W1.transpose(0, 2, 1, 3).reshape(L, DM, E * DFF)
    b1 = e_b1.reshape(L, 1, E * DFF)
    b2 = e_b2.reshape(L, E, 1, DM)

    sw = te_scale_W.reshape(1, 1)
    sb = te_scale_b.reshape(1, 1)
    pw = jnp.concatenate([jnp.zeros((1, 1), _F), te_per_W], axis=1)
    pb = jnp.concatenate([jnp.zeros((1, 1), _F),
                          te_per_b.reshape(1, DM - 1)], axis=1)

    db1 = dec_b1.reshape(1, DM)
    db2 = dec_b2.reshape(1, DM)
    db3 = dec_b3.reshape(1, 1)

    def full(shape):
        return pl.BlockSpec(shape, lambda i: (0,) * len(shape))

    o = pl.pallas_call(
        _moe_dec_kernel,
        grid=(B,),
        in_specs=[
            pl.BlockSpec((1, RT, 1), lambda i: (i, 0, 0)),
            pl.BlockSpec((1, LPRED, 1), lambda i: (i, 0, 0)),
            full((1, DM)), full((1, DM)),
            full((L, DM, EP)), full((1, EP)),
            full((L, DM, E * DFF)), full((L, 1, E * DFF)),
            full((L, E, DFF, DM)), full((L, E, 1, DM)),
            full((1, 1)), full((1, 1)), full((1, DM)), full((1, DM)),
            full((2 * DM, DM)), full((1, DM)),
            full((DM, DM)), full((1, DM)),
            full((DM, 1)), full((1, 1)),
        ],
        out_specs=pl.BlockSpec((1, RD, 1), lambda i: (i, 0, 0)),
        out_shape=jax.ShapeDtypeStruct((B, RD, 1), _F),
    )(x, tt, W_start, b_start.reshape(1, DM), gw, gb, w1, b1, e_W2, b2,
      sw, sb, pw, pb, dec_W1, db1, dec_W2, db2, dec_W3, db3)

    return o.reshape(B, LPRED, NP_)[:, :, :N][None]


# speedup vs baseline: 1.0001x; 1.0001x over previous
import jax
import jax.numpy as jnp
from jax.experimental import pallas as pl
from jax.experimental.pallas import tpu as pltpu

B = 32
OBS = 72
SEQ = 96
N = 21
NP_ = 24
DM = 128
DFF = 256
L = 2
E = 4
EP = 8
K = 2
LPRED = 96
RT = NP_ * SEQ
RD = LPRED * NP_

_F = jnp.float32
_HI = jax.lax.Precision.HIGHEST


def _moe_dec_kernel(x_ref, tt_ref, wstart_ref, bstart_ref, gw_ref, gb_ref,
                    w1_ref, b1_ref, w2_ref, b2_ref,
                    sw_ref, sb_ref, pw_ref, pb_ref,
                    dw1_ref, db1_ref, dw2_ref, db2_ref,
                    dw3_ref, db3_ref,
                    out_ref):
    x = x_ref[0]
    tok = x * wstart_ref[...] + bstart_ref[...]

    for l in range(L):
        logits = jnp.dot(tok, gw_ref[l], preferred_element_type=_F) + gb_ref[...]
        lane = jax.lax.broadcasted_iota(jnp.int32, (RT, EP), 1)
        m1 = jnp.max(logits, axis=1, keepdims=True)
        i1 = jnp.min(jnp.where(logits == m1, lane, EP), axis=1, keepdims=True)
        is1 = lane == i1
        l2 = jnp.where(is1, -1e30, logits)
        m2 = jnp.max(l2, axis=1, keepdims=True)
        i2 = jnp.min(jnp.where(l2 == m2, lane, EP), axis=1, keepdims=True)
        is2 = lane == i2
        ex = jnp.exp(m2 - m1)
        den = 1.0 + ex
        gates = ((1.0 / den) * is1.astype(_F)
                 + (ex / den) * is2.astype(_F))

        h = jnp.maximum(jnp.dot(tok, w1_ref[l], preferred_element_type=_F)
                        + b1_ref[l], 0.0)
        y = None
        for e in range(E):
            ye = jnp.dot(h[:, e * DFF:(e + 1) * DFF], w2_ref[l, e],
                         preferred_element_type=_F) + b2_ref[l, e]
            ye = gates[:, e:e + 1] * ye
            y = ye if y is None else y + ye
        tok = tok + y

    enc = jnp.sum(tok.reshape(NP_, SEQ, DM), axis=1) * (1.0 / SEQ)

    tt = tt_ref[0]
    lane = jax.lax.broadcasted_iota(jnp.int32, (LPRED, DM), 1)
    te = jnp.where(lane == 0, tt * sw_ref[...] + sb_ref[...],
                   jnp.sin(tt * pw_ref[...] + pb_ref[...]))

    encf = jnp.concatenate(
        [jnp.broadcast_to(enc[None, :, :], (LPRED, NP_, DM)).reshape(RD, DM),
         jnp.broadcast_to(te[:, None, :], (LPRED, NP_, DM)).reshape(RD, DM)],
        axis=1)
    h1 = jnp.maximum(jnp.dot(encf, dw1_ref[...], preferred_element_type=_F)
                     + db1_ref[...], 0.0)
    h2 = jnp.maximum(jnp.dot(h1, dw2_ref[...], preferred_element_type=_F)
                     + db2_ref[...], 0.0)
    o = (jnp.dot(h2, dw3_ref[...], preferred_element_type=_F)
         + db3_ref[...])
    out_ref[0] = o


def kernel(tp_to_predict, observed_data, observed_tp, observed_mask, W_start,
           b_start, gate_W, e_W1, e_b1, e_W2, e_b2, te_scale_W, te_scale_b,
           te_per_W, te_per_b, dec_W1, dec_b1, dec_W2, dec_b2, dec_W3, dec_b3):
    x = jnp.pad(observed_data, ((0, 0), (0, SEQ - OBS), (0, 0)))
    x = jnp.pad(x.transpose(0, 2, 1), ((0, 0), (0, NP_ - N), (0, 0)))
    x = x.reshape(B, RT, 1)
    tt = tp_to_predict.reshape(B, LPRED, 1)

    gw = jnp.concatenate([gate_W, jnp.zeros((L, DM, EP - E), _F)], axis=2)
    gb = jnp.concatenate([jnp.zeros((1, E), _F),
                          jnp.full((1, EP - E), -1e30, _F)], axis=1)
    w1 = e_W1.transpose(0, 2, 1, 3).reshape(L, DM, E * DFF)
    b1 = e_b1.reshape(L, 1, E * DFF)
    b2 = e_b2.reshape(L, E, 1, DM)

    sw = te_scale_W.reshape(1, 1)
    sb = te_scale_b.reshape(1, 1)
    pw = jnp.concatenate([jnp.zeros((1, 1), _F), te_per_W], axis=1)
    pb = jnp.concatenate([jnp.zeros((1, 1), _F),
                          te_per_b.reshape(1, DM - 1)], axis=1)

    db1 = dec_b1.reshape(1, DM)
    db2 = dec_b2.reshape(1, DM)
    db3 = dec_b3.reshape(1, 1)

    def full(shape):
        return pl.BlockSpec(shape, lambda i: (0,) * len(shape))

    o = pl.pallas_call(
        _moe_dec_kernel,
        grid=(B,),
        in_specs=[
            pl.BlockSpec((1, RT, 1), lambda i: (i, 0, 0)),
            pl.BlockSpec((1, LPRED, 1), lambda i: (i, 0, 0)),
            full((1, DM)), full((1, DM)),
            full((L, DM, EP)), full((1, EP)),
            full((L, DM, E * DFF)), full((L, 1, E * DFF)),
            full((L, E, DFF, DM)), full((L, E, 1, DM)),
            full((1, 1)), full((1, 1)), full((1, DM)), full((1, DM)),
            full((2 * DM, DM)), full((1, DM)),
            full((DM, DM)), full((1, DM)),
            full((DM, 1)), full((1, 1)),
        ],
        out_specs=pl.BlockSpec((1, RD, 1), lambda i: (i, 0, 0)),
        compiler_params=pltpu.CompilerParams(
            dimension_semantics=("parallel",)),
        out_shape=jax.ShapeDtypeStruct((B, RD, 1), _F),
    )(x, tt, W_start, b_start.reshape(1, DM), gw, gb, w1, b1, e_W2, b2,
      sw, sb, pw, pb, dec_W1, db1, dec_W2, db2, dec_W3, db3)

    return o.reshape(B, LPRED, NP_)[:, :, :N][None]


# speedup vs baseline: 1.1527x; 1.1525x over previous
import jax
import jax.numpy as jnp
from jax.experimental import pallas as pl
from jax.experimental.pallas import tpu as pltpu

B = 32
OBS = 72
SEQ = 96
N = 21
NP_ = 24
DM = 128
DFF = 256
L = 2
E = 4
EP = 8
K = 2
LPRED = 96
RT = NP_ * SEQ
RD = LPRED * NP_

_F = jnp.float32
_HI = jax.lax.Precision.HIGHEST


def _moe_dec_kernel(x_ref, tt_ref, wstart_ref, bstart_ref, gw_ref, gb_ref,
                    w1_ref, b1_ref, w2_ref, b2_ref,
                    sw_ref, sb_ref, pw_ref, pb_ref,
                    dw1_ref, db1_ref, dw2_ref, db2_ref,
                    dw3_ref, db3_ref,
                    out_ref):
    x = x_ref[0]
    tok = x * wstart_ref[...] + bstart_ref[...]

    for l in range(L):
        logits = jnp.dot(tok, gw_ref[l], preferred_element_type=_F) + gb_ref[...]
        m1 = jnp.max(logits, axis=1, keepdims=True)
        is1 = logits == m1
        l2 = jnp.where(is1, -1e30, logits)
        m2 = jnp.max(l2, axis=1, keepdims=True)
        is2 = l2 == m2
        ex = jnp.exp(m2 - m1)
        den = 1.0 + ex
        gates = ((1.0 / den) * is1.astype(_F)
                 + (ex / den) * is2.astype(_F))

        h = jnp.maximum(jnp.dot(tok, w1_ref[l], preferred_element_type=_F)
                        + b1_ref[l], 0.0)
        y = None
        for e in range(E):
            ye = jnp.dot(h[:, e * DFF:(e + 1) * DFF], w2_ref[l, e],
                         preferred_element_type=_F) + b2_ref[l, e]
            ye = gates[:, e:e + 1] * ye
            y = ye if y is None else y + ye
        tok = tok + y

    enc = jnp.sum(tok.reshape(NP_, SEQ, DM), axis=1) * (1.0 / SEQ)

    tt = tt_ref[0]
    lane = jax.lax.broadcasted_iota(jnp.int32, (LPRED, DM), 1)
    te = jnp.where(lane == 0, tt * sw_ref[...] + sb_ref[...],
                   jnp.sin(tt * pw_ref[...] + pb_ref[...]))

    encf = jnp.concatenate(
        [jnp.broadcast_to(enc[None, :, :], (LPRED, NP_, DM)).reshape(RD, DM),
         jnp.broadcast_to(te[:, None, :], (LPRED, NP_, DM)).reshape(RD, DM)],
        axis=1)
    h1 = jnp.maximum(jnp.dot(encf, dw1_ref[...], preferred_element_type=_F)
                     + db1_ref[...], 0.0)
    h2 = jnp.maximum(jnp.dot(h1, dw2_ref[...], preferred_element_type=_F)
                     + db2_ref[...], 0.0)
    o = (jnp.dot(h2, dw3_ref[...], preferred_element_type=_F)
         + db3_ref[...])
    out_ref[0] = o


def kernel(tp_to_predict, observed_data, observed_tp, observed_mask, W_start,
           b_start, gate_W, e_W1, e_b1, e_W2, e_b2, te_scale_W, te_scale_b,
           te_per_W, te_per_b, dec_W1, dec_b1, dec_W2, dec_b2, dec_W3, dec_b3):
    x = jnp.pad(observed_data, ((0, 0), (0, SEQ - OBS), (0, 0)))
    x = jnp.pad(x.transpose(0, 2, 1), ((0, 0), (0, NP_ - N), (0, 0)))
    x = x.reshape(B, RT, 1)
    tt = tp_to_predict.reshape(B, LPRED, 1)

    gw = jnp.concatenate([gate_W, jnp.zeros((L, DM, EP - E), _F)], axis=2)
    gb = jnp.concatenate([jnp.zeros((1, E), _F),
                          jnp.full((1, EP - E), -1e30, _F)], axis=1)
    w1 = e_W1.transpose(0, 2, 1, 3).reshape(L, DM, E * DFF)
    b1 = e_b1.reshape(L, 1, E * DFF)
    b2 = e_b2.reshape(L, E, 1, DM)

    sw = te_scale_W.reshape(1, 1)
    sb = te_scale_b.reshape(1, 1)
    pw = jnp.concatenate([jnp.zeros((1, 1), _F), te_per_W], axis=1)
    pb = jnp.concatenate([jnp.zeros((1, 1), _F),
                          te_per_b.reshape(1, DM - 1)], axis=1)

    db1 = dec_b1.reshape(1, DM)
    db2 = dec_b2.reshape(1, DM)
    db3 = dec_b3.reshape(1, 1)

    def full(shape):
        return pl.BlockSpec(shape, lambda i: (0,) * len(shape))

    o = pl.pallas_call(
        _moe_dec_kernel,
        grid=(B,),
        in_specs=[
            pl.BlockSpec((1, RT, 1), lambda i: (i, 0, 0)),
            pl.BlockSpec((1, LPRED, 1), lambda i: (i, 0, 0)),
            full((1, DM)), full((1, DM)),
            full((L, DM, EP)), full((1, EP)),
            full((L, DM, E * DFF)), full((L, 1, E * DFF)),
            full((L, E, DFF, DM)), full((L, E, 1, DM)),
            full((1, 1)), full((1, 1)), full((1, DM)), full((1, DM)),
            full((2 * DM, DM)), full((1, DM)),
            full((DM, DM)), full((1, DM)),
            full((DM, 1)), full((1, 1)),
        ],
        out_specs=pl.BlockSpec((1, RD, 1), lambda i: (i, 0, 0)),
        compiler_params=pltpu.CompilerParams(
            dimension_semantics=("parallel",)),
        out_shape=jax.ShapeDtypeStruct((B, RD, 1), _F),
    )(x, tt, W_start, b_start.reshape(1, DM), gw, gb, w1, b1, e_W2, b2,
      sw, sb, pw, pb, dec_W1, db1, dec_W2, db2, dec_W3, db3)

    return o.reshape(B, LPRED, NP_)[:, :, :N][None]
